# baseline (speedup 1.0000x reference)
import functools

import jax
import jax.numpy as jnp
from jax import lax
from jax.experimental import pallas as pl
from jax.experimental.pallas import tpu as pltpu
from jax.experimental.pallas import tpu_sc as plsc

HOURS = 24 * 7
SU, SL, TU, TL = 100.0, 0.0, 1000.0, 0.0
B, L, D = 1024, 20, 64
LL = L * L
D2 = 2 * D

NC, NS = 2, 16
NW = NC * NS
ROWS_W = (B * L) // NW
CH = 128
NCH = ROWS_W // CH
USR_W = B // NW


def _sc_gather_body(traj_hbm, tim_hbm, user_hbm, embl_hbm, embt_hbm, embu_hbm,
                    out_l, out_t, out_u,
                    idx_l, idx_t, idx_u, rows, rows_u, sem):
    wid = lax.axis_index("s") * NC + lax.axis_index("c")
    ubase = wid * USR_W

    pltpu.sync_copy(traj_hbm.at[wid], idx_l)
    pltpu.sync_copy(tim_hbm.at[wid], idx_t)
    pltpu.sync_copy(user_hbm.at[pl.ds(ubase, USR_W)], idx_u)

    for j in range(NCH):
        for k in range(CH // 16):
            sl = pl.ds(k * 16, 16)
            v = idx_l[j, sl]
            idx_l[j, sl] = v >> 1
            t = idx_t[j, sl]
            t2 = jnp.where(t == 0, HOURS, t)
            idx_t[j, sl] = t2 >> 1
    for k in range(USR_W // 16):
        sl = pl.ds(k * 16, 16)
        idx_u[sl] = idx_u[sl] >> 1

    copies = [pltpu.async_copy(
        embl_hbm.at[idx_l.at[j]], rows.at[pl.ds(j * CH, CH)], sem)
        for j in range(NCH)]
    for c in copies:
        c.wait()
    pltpu.sync_copy(rows, out_l.at[pl.ds(wid * ROWS_W, ROWS_W)])

    copies = [pltpu.async_copy(
        embt_hbm.at[idx_t.at[j]], rows.at[pl.ds(j * CH, CH)], sem)
        for j in range(NCH)]
    for c in copies:
        c.wait()
    pltpu.sync_copy(rows, out_t.at[pl.ds(wid * ROWS_W, ROWS_W)])

    pltpu.async_copy(embu_hbm.at[idx_u], rows_u, sem).wait()
    pltpu.sync_copy(rows_u, out_u.at[pl.ds(ubase, USR_W)])


@functools.cache
def _sc_gather_kernel():
    mesh = plsc.VectorSubcoreMesh(
        core_axis_name="c", subcore_axis_name="s",
        num_cores=NC, num_subcores=NS)
    return pl.kernel(
        _sc_gather_body,
        mesh=mesh,
        out_type=(
            jax.ShapeDtypeStruct((B * L, D2), jnp.float32),
            jax.ShapeDtypeStruct((B * L, D2), jnp.float32),
            jax.ShapeDtypeStruct((B, D2), jnp.float32),
        ),
        scratch_types=[
            pltpu.VMEM((NCH, CH), jnp.int32),
            pltpu.VMEM((NCH, CH), jnp.int32),
            pltpu.VMEM((USR_W,), jnp.int32),
            pltpu.VMEM((ROWS_W, D2), jnp.float32),
            pltpu.VMEM((USR_W, D2), jnp.float32),
            pltpu.SemaphoreType.DMA,
        ],
        compiler_params=pltpu.CompilerParams(use_tc_tiling_on_sc=True),
    )


I_BLK = 2


def _delta_body(tl_ref, mat_ref, esl_ref, esu_ref, etl_ref, etu_ref, out_ref):
    i0 = pl.program_id(0) * I_BLK
    tl = tl_ref[...]
    ii = i0 + lax.broadcasted_iota(jnp.int32, (I_BLK, L, 1, 1), 0)
    jj = lax.broadcasted_iota(jnp.int32, (I_BLK, L, 1, 1), 1)
    m = (tl > ii) & (tl > jj)

    esl = esl_ref[...]
    esu = esu_ref[...]
    etl = etl_ref[...]
    etu = etu_ref[...]
    inv_s = 1.0 / (SU - SL)
    inv_t = 1.0 / (TU - TL)
    a = (esu - esl) * inv_s
    b = (etu - etl) * inv_t
    c = (esl * SU - esu * SL) * inv_s + (etl * TU - etu * TL) * inv_t

    wa = jnp.where(m, a[:, :, :, 1:2], a[:, :, :, 0:1])
    wb = jnp.where(m, b[:, :, :, 1:2], b[:, :, :, 0:1])
    wc = jnp.where(m, c[:, :, :, 1:2], c[:, :, :, 0:1])

    ds = mat_ref[:, :, 0:1, :]
    dt = mat_ref[:, :, 1:2, :]
    out_ref[...] = wa * ds + wb * dt + wc


_full4 = lambda shape: pl.BlockSpec(shape, lambda i: (0, 0, 0, 0))

_tc_delta = pl.pallas_call(
    _delta_body,
    grid=(L // I_BLK,),
    in_specs=[
        _full4((1, 1, 1, B)),
        pl.BlockSpec((I_BLK, L, 2, B), lambda i: (i, 0, 0, 0)),
        _full4((1, 1, D, 2)), _full4((1, 1, D, 2)),
        _full4((1, 1, D, 2)), _full4((1, 1, D, 2)),
    ],
    out_specs=pl.BlockSpec((I_BLK, L, D, B), lambda i: (i, 0, 0, 0)),
    out_shape=jax.ShapeDtypeStruct((L, L, D, B), jnp.float32),
    compiler_params=pltpu.CompilerParams(
        dimension_semantics=("arbitrary",)),
)


BBJ = 32


def _half_select(pair, parity):
    lo = pair[..., :D]
    hi = pair[..., D:]
    return jnp.where(parity == 1, hi, lo)


def _joint_body(traj_ref, tim_ref, usr_ref, rl_ref, rt_ref, ru_ref, joint_ref):
    loc_e = _half_select(rl_ref[...], traj_ref[...] & 1)
    tim = tim_ref[...]
    tim2 = jnp.where(tim == 0, HOURS, tim)
    time_e = _half_select(rt_ref[...], tim2 & 1)
    usr_e = _half_select(ru_ref[...], usr_ref[...] & 1)
    joint_ref[...] = loc_e + time_e + usr_e[:, None, :]


_tc_joint = pl.pallas_call(
    _joint_body,
    grid=(B // BBJ,),
    in_specs=[
        pl.BlockSpec((BBJ, L, 1), lambda i: (i, 0, 0)),
        pl.BlockSpec((BBJ, L, 1), lambda i: (i, 0, 0)),
        pl.BlockSpec((BBJ, 1), lambda i: (i, 0)),
        pl.BlockSpec((BBJ, L, D2), lambda i: (i, 0, 0)),
        pl.BlockSpec((BBJ, L, D2), lambda i: (i, 0, 0)),
        pl.BlockSpec((BBJ, D2), lambda i: (i, 0)),
    ],
    out_specs=pl.BlockSpec((BBJ, L, D), lambda i: (i, 0, 0)),
    out_shape=jax.ShapeDtypeStruct((B, L, D), jnp.float32),
    compiler_params=pltpu.CompilerParams(
        dimension_semantics=("arbitrary",)),
)


def kernel(user, tim, traj, mat, traj_len, emb_t, emb_l, emb_u,
           emb_su, emb_sl, emb_tu, emb_tl):
    traj3d = traj.astype(jnp.int32).reshape(NW, NCH, CH)
    tim3d = tim.astype(jnp.int32).reshape(NW, NCH, CH)
    user_i = user.astype(jnp.int32)

    mat_p = jnp.transpose(mat, (1, 2, 3, 0))
    tl4 = traj_len.astype(jnp.int32).reshape(1, 1, 1, B)
    esl_p = emb_sl.T.reshape(1, 1, D, 2)
    esu_p = emb_su.T.reshape(1, 1, D, 2)
    etl_p = emb_tl.T.reshape(1, 1, D, 2)
    etu_p = emb_tu.T.reshape(1, 1, D, 2)
    delta_p = _tc_delta(tl4, mat_p, esl_p, esu_p, etl_p, etu_p)
    delta = jnp.transpose(delta_p, (3, 0, 1, 2))

    embl2 = emb_l.reshape(emb_l.shape[0] // 2, D2)
    embt2 = jnp.pad(emb_t, ((0, 1), (0, 0))).reshape((emb_t.shape[0] + 1) // 2, D2)
    embu2 = emb_u.reshape(emb_u.shape[0] // 2, D2)

    rows_l, rows_t, rows_u = _sc_gather_kernel()(
        traj3d, tim3d, user_i, embl2, embt2, embu2)

    joint = _tc_joint(
        traj.astype(jnp.int32).reshape(B, L, 1),
        tim.astype(jnp.int32).reshape(B, L, 1),
        user_i.reshape(B, 1),
        rows_l.reshape(B, L, D2), rows_t.reshape(B, L, D2), rows_u)

    return joint, delta


# speedup vs baseline: 1.2709x; 1.2709x over previous
import functools

import jax
import jax.numpy as jnp
from jax import lax
from jax.experimental import pallas as pl
from jax.experimental.pallas import tpu as pltpu
from jax.experimental.pallas import tpu_sc as plsc

HOURS = 24 * 7
SU, SL, TU, TL = 100.0, 0.0, 1000.0, 0.0
B, L, D = 1024, 20, 64
LL = L * L
G = 8

NC, NS = 2, 16
NW = NC * NS
ROWS_W = (B * L) // NW
USR_W = B // NW
K = 16


def _gather_rows(tbl, idx_ref, n, remap, gring, rows, sem):
    def _load_idx(gbase):
        vs = idx_ref[pl.ds(gbase, K)]
        if remap:
            vs = jnp.where(vs == 0, HOURS, vs)
        return vs

    def _issue_group(slot_base, gbase):
        grp_v = (_load_idx(gbase) >> 3) * G
        for b in range(K):
            grp = pl.multiple_of(grp_v[b], G)
            pltpu.async_copy(tbl.at[pl.ds(grp, G)],
                             gring.at[slot_base + b], sem)

    def _drain_group(slot_base, gbase):
        s_v = _load_idx(gbase) & 7
        for b in range(K):
            pltpu.make_async_copy(
                tbl.at[pl.ds(0, G)], gring.at[slot_base + b], sem).wait()
            s = s_v[b]
            for c in range(D // 16):
                sl = pl.ds(c * 16, 16)
                rows[gbase + b, sl] = gring[slot_base + b, s, sl]

    _issue_group(0, 0)

    def body(t, carry):
        g = t * K
        half = t & 1
        _issue_group((1 - half) * K, g + K)
        _drain_group(half * K, g)
        return carry

    lax.fori_loop(0, n // K - 1, body, 0, unroll=False)
    _drain_group(((n // K - 1) & 1) * K, n - K)


def _sc_gather_body(traj_hbm, tim_hbm, user_hbm, embl_hbm, embt_hbm, embu_hbm,
                    out_l, out_t, out_u,
                    idx_v, gring, rows, rows_u, sem):
    wid = lax.axis_index("s") * NC + lax.axis_index("c")
    ubase = wid * USR_W

    pltpu.sync_copy(traj_hbm.at[wid], idx_v)
    _gather_rows(embl_hbm, idx_v.at[0], ROWS_W, False, gring, rows, sem)
    pltpu.sync_copy(rows, out_l.at[pl.ds(wid * ROWS_W, ROWS_W)])

    pltpu.sync_copy(tim_hbm.at[wid], idx_v)
    _gather_rows(embt_hbm, idx_v.at[0], ROWS_W, True, gring, rows, sem)
    pltpu.sync_copy(rows, out_t.at[pl.ds(wid * ROWS_W, ROWS_W)])

    pltpu.sync_copy(user_hbm.at[pl.ds(ubase, USR_W)],
                    idx_v.at[0, pl.ds(0, USR_W)])
    _gather_rows(embu_hbm, idx_v.at[0], USR_W, False, gring, rows_u, sem)
    pltpu.sync_copy(rows_u, out_u.at[pl.ds(ubase, USR_W)])


@functools.cache
def _sc_gather_kernel():
    mesh = plsc.VectorSubcoreMesh(
        core_axis_name="c", subcore_axis_name="s",
        num_cores=NC, num_subcores=NS)
    return pl.kernel(
        _sc_gather_body,
        mesh=mesh,
        out_type=(
            jax.ShapeDtypeStruct((B * L, D), jnp.float32),
            jax.ShapeDtypeStruct((B * L, D), jnp.float32),
            jax.ShapeDtypeStruct((B, D), jnp.float32),
        ),
        scratch_types=[
            pltpu.VMEM((1, ROWS_W), jnp.int32),
            pltpu.VMEM((2 * K, G, D), jnp.float32),
            pltpu.VMEM((ROWS_W, D), jnp.float32),
            pltpu.VMEM((USR_W, D), jnp.float32),
            pltpu.SemaphoreType.DMA,
        ],
        compiler_params=pltpu.CompilerParams(use_tc_tiling_on_sc=True),
    )


I_BLK = 2


def _delta_body(tl_ref, mat_ref, esl_ref, esu_ref, etl_ref, etu_ref, out_ref):
    i0 = pl.program_id(0) * I_BLK
    tl = tl_ref[...]
    ii = i0 + lax.broadcasted_iota(jnp.int32, (I_BLK, L, 1, 1), 0)
    jj = lax.broadcasted_iota(jnp.int32, (I_BLK, L, 1, 1), 1)
    m = (tl > ii) & (tl > jj)

    esl = esl_ref[...]
    esu = esu_ref[...]
    etl = etl_ref[...]
    etu = etu_ref[...]
    inv_s = 1.0 / (SU - SL)
    inv_t = 1.0 / (TU - TL)
    a = (esu - esl) * inv_s
    b = (etu - etl) * inv_t
    c = (esl * SU - esu * SL) * inv_s + (etl * TU - etu * TL) * inv_t

    wa = jnp.where(m, a[:, :, :, 1:2], a[:, :, :, 0:1])
    wb = jnp.where(m, b[:, :, :, 1:2], b[:, :, :, 0:1])
    wc = jnp.where(m, c[:, :, :, 1:2], c[:, :, :, 0:1])

    ds = mat_ref[:, :, 0:1, :]
    dt = mat_ref[:, :, 1:2, :]
    out_ref[...] = wa * ds + wb * dt + wc


_full4 = lambda shape: pl.BlockSpec(shape, lambda i: (0, 0, 0, 0))

_tc_delta = pl.pallas_call(
    _delta_body,
    grid=(L // I_BLK,),
    in_specs=[
        _full4((1, 1, 1, B)),
        pl.BlockSpec((I_BLK, L, 2, B), lambda i: (i, 0, 0, 0)),
        _full4((1, 1, D, 2)), _full4((1, 1, D, 2)),
        _full4((1, 1, D, 2)), _full4((1, 1, D, 2)),
    ],
    out_specs=pl.BlockSpec((I_BLK, L, D, B), lambda i: (i, 0, 0, 0)),
    out_shape=jax.ShapeDtypeStruct((L, L, D, B), jnp.float32),
    compiler_params=pltpu.CompilerParams(
        dimension_semantics=("arbitrary",)),
)


BBJ = 32


def _joint_body(rl_ref, rt_ref, ru_ref, joint_ref):
    joint_ref[...] = rl_ref[...] + rt_ref[...] + ru_ref[...][:, None, :]


_tc_joint = pl.pallas_call(
    _joint_body,
    grid=(B // BBJ,),
    in_specs=[
        pl.BlockSpec((BBJ, L, D), lambda i: (i, 0, 0)),
        pl.BlockSpec((BBJ, L, D), lambda i: (i, 0, 0)),
        pl.BlockSpec((BBJ, D), lambda i: (i, 0)),
    ],
    out_specs=pl.BlockSpec((BBJ, L, D), lambda i: (i, 0, 0)),
    out_shape=jax.ShapeDtypeStruct((B, L, D), jnp.float32),
    compiler_params=pltpu.CompilerParams(
        dimension_semantics=("arbitrary",)),
)


def kernel(user, tim, traj, mat, traj_len, emb_t, emb_l, emb_u,
           emb_su, emb_sl, emb_tu, emb_tl):
    traj3d = traj.astype(jnp.int32).reshape(NW, 1, ROWS_W)
    tim3d = tim.astype(jnp.int32).reshape(NW, 1, ROWS_W)
    user_i = user.astype(jnp.int32)

    mat_p = jnp.transpose(mat, (1, 2, 3, 0))
    tl4 = traj_len.astype(jnp.int32).reshape(1, 1, 1, B)
    esl_p = emb_sl.T.reshape(1, 1, D, 2)
    esu_p = emb_su.T.reshape(1, 1, D, 2)
    etl_p = emb_tl.T.reshape(1, 1, D, 2)
    etu_p = emb_tu.T.reshape(1, 1, D, 2)
    delta_p = _tc_delta(tl4, mat_p, esl_p, esu_p, etl_p, etu_p)
    delta = jnp.transpose(delta_p, (3, 0, 1, 2))

    embt_pad = jnp.pad(emb_t, ((0, 7), (0, 0)))

    rows_l, rows_t, rows_u = _sc_gather_kernel()(
        traj3d, tim3d, user_i, emb_l, embt_pad, emb_u)

    joint = _tc_joint(
        rows_l.reshape(B, L, D), rows_t.reshape(B, L, D), rows_u)

    return joint, delta


# speedup vs baseline: 1.2741x; 1.0025x over previous
import functools

import jax
import jax.numpy as jnp
from jax import lax
from jax.experimental import pallas as pl
from jax.experimental.pallas import tpu as pltpu
from jax.experimental.pallas import tpu_sc as plsc

HOURS = 24 * 7
SU, SL, TU, TL = 100.0, 0.0, 1000.0, 0.0
B, L, D = 1024, 20, 64
LL = L * L
G = 8

NC, NS = 2, 16
NW = NC * NS
ROWS_W = (B * L) // NW
USR_W = B // NW
K = 32


def _gather_rows(tbl, idx_ref, n, remap, gring, rows, out, out_base,
                 semg, semo):
    ngrp = n // K

    def _load_idx(gbase):
        out = []
        for q in range(K // 16):
            vs = idx_ref[pl.ds(gbase + q * 16, 16)]
            if remap:
                vs = jnp.where(vs == 0, HOURS, vs)
            out.append(vs)
        return out

    def _issue_group(slot_base, gbase):
        grp_v = [(v >> 3) * G for v in _load_idx(gbase)]
        for b in range(K):
            grp = pl.multiple_of(grp_v[b // 16][b % 16], G)
            pltpu.async_copy(tbl.at[pl.ds(grp, G)],
                             gring.at[slot_base + b], semg)

    def _extract_group(slot_base, gbase, half):
        s_v = [v & 7 for v in _load_idx(gbase)]
        for b in range(K):
            pltpu.make_async_copy(
                tbl.at[pl.ds(0, G)], gring.at[slot_base + b], semg).wait()
            s = s_v[b // 16][b % 16]
            for c in range(D // 16):
                sl = pl.ds(c * 16, 16)
                rows[half * K + b, sl] = gring[slot_base + b, s, sl]

    def _out_drain():
        pltpu.make_async_copy(rows.at[pl.ds(0, K)],
                              out.at[pl.ds(out_base, K)], semo).wait()

    _issue_group(0, 0)

    def body(t, carry):
        half = t & 1

        @pl.when(t + 1 < ngrp)
        def _():
            _issue_group((1 - half) * K, (t + 1) * K)

        @pl.when(t >= 2)
        def _():
            _out_drain()

        _extract_group(half * K, t * K, half)
        pltpu.async_copy(rows.at[pl.ds(half * K, K)],
                         out.at[pl.ds(out_base + t * K, K)], semo)
        return carry

    lax.fori_loop(0, ngrp, body, 0, unroll=False)
    for _ in range(min(2, ngrp)):
        _out_drain()


def _sc_loc_body(traj_hbm, embl_hbm, out_l, idx_v, gring, rows, semg, semo):
    wid = lax.axis_index("s") * NC + lax.axis_index("c")
    pltpu.sync_copy(traj_hbm.at[wid], idx_v)
    _gather_rows(embl_hbm, idx_v.at[0], ROWS_W, False, gring, rows,
                 out_l, wid * ROWS_W, semg, semo)


def _sc_tu_body(tim_hbm, user_hbm, embt_hbm, embu_hbm, out_t, out_u,
                idx_v, gring, rows, semg, semo):
    wid = lax.axis_index("s") * NC + lax.axis_index("c")
    ubase = wid * USR_W

    pltpu.sync_copy(tim_hbm.at[wid], idx_v)
    _gather_rows(embt_hbm, idx_v.at[0], ROWS_W, True, gring, rows,
                 out_t, wid * ROWS_W, semg, semo)

    pltpu.sync_copy(user_hbm.at[pl.ds(ubase, USR_W)],
                    idx_v.at[0, pl.ds(0, USR_W)])
    _gather_rows(embu_hbm, idx_v.at[0], USR_W, False, gring, rows,
                 out_u, ubase, semg, semo)


@functools.cache
def _sc_kernels():
    mesh = plsc.VectorSubcoreMesh(
        core_axis_name="c", subcore_axis_name="s",
        num_cores=NC, num_subcores=NS)
    loc = pl.kernel(
        _sc_loc_body,
        mesh=mesh,
        out_type=jax.ShapeDtypeStruct((B * L, D), jnp.float32),
        scratch_types=[
            pltpu.VMEM((1, ROWS_W), jnp.int32),
            pltpu.VMEM((2 * K, G, D), jnp.float32),
            pltpu.VMEM((2 * K, D), jnp.float32),
            pltpu.SemaphoreType.DMA,
            pltpu.SemaphoreType.DMA,
        ],
        compiler_params=pltpu.CompilerParams(use_tc_tiling_on_sc=True),
    )
    tu = pl.kernel(
        _sc_tu_body,
        mesh=mesh,
        out_type=(
            jax.ShapeDtypeStruct((B * L, D), jnp.float32),
            jax.ShapeDtypeStruct((B, D), jnp.float32),
        ),
        scratch_types=[
            pltpu.VMEM((1, ROWS_W), jnp.int32),
            pltpu.VMEM((2 * K, G, D), jnp.float32),
            pltpu.VMEM((2 * K, D), jnp.float32),
            pltpu.SemaphoreType.DMA,
            pltpu.SemaphoreType.DMA,
        ],
        compiler_params=pltpu.CompilerParams(use_tc_tiling_on_sc=True),
    )
    return loc, tu


I_BLK = 2


def _delta_body(tl_ref, mat_ref, esl_ref, esu_ref, etl_ref, etu_ref, out_ref):
    i0 = pl.program_id(0) * I_BLK
    tl = tl_ref[...]
    ii = i0 + lax.broadcasted_iota(jnp.int32, (I_BLK, L, 1, 1), 0)
    jj = lax.broadcasted_iota(jnp.int32, (I_BLK, L, 1, 1), 1)
    m = (tl > ii) & (tl > jj)

    esl = esl_ref[...]
    esu = esu_ref[...]
    etl = etl_ref[...]
    etu = etu_ref[...]
    inv_s = 1.0 / (SU - SL)
    inv_t = 1.0 / (TU - TL)
    a = (esu - esl) * inv_s
    b = (etu - etl) * inv_t
    c = (esl * SU - esu * SL) * inv_s + (etl * TU - etu * TL) * inv_t

    wa = jnp.where(m, a[:, :, :, 1:2], a[:, :, :, 0:1])
    wb = jnp.where(m, b[:, :, :, 1:2], b[:, :, :, 0:1])
    wc = jnp.where(m, c[:, :, :, 1:2], c[:, :, :, 0:1])

    ds = mat_ref[:, :, 0:1, :]
    dt = mat_ref[:, :, 1:2, :]
    out_ref[...] = wa * ds + wb * dt + wc


_full4 = lambda shape: pl.BlockSpec(shape, lambda i: (0, 0, 0, 0))

_tc_delta = pl.pallas_call(
    _delta_body,
    grid=(L // I_BLK,),
    in_specs=[
        _full4((1, 1, 1, B)),
        pl.BlockSpec((I_BLK, L, 2, B), lambda i: (i, 0, 0, 0)),
        _full4((1, 1, D, 2)), _full4((1, 1, D, 2)),
        _full4((1, 1, D, 2)), _full4((1, 1, D, 2)),
    ],
    out_specs=pl.BlockSpec((I_BLK, L, D, B), lambda i: (i, 0, 0, 0)),
    out_shape=jax.ShapeDtypeStruct((L, L, D, B), jnp.float32),
    compiler_params=pltpu.CompilerParams(
        dimension_semantics=("arbitrary",)),
)


BBJ = 32


def _joint_body(rl_ref, rt_ref, ru_ref, joint_ref):
    joint_ref[...] = rl_ref[...] + rt_ref[...] + ru_ref[...][:, None, :]


_tc_joint = pl.pallas_call(
    _joint_body,
    grid=(B // BBJ,),
    in_specs=[
        pl.BlockSpec((BBJ, L, D), lambda i: (i, 0, 0)),
        pl.BlockSpec((BBJ, L, D), lambda i: (i, 0, 0)),
        pl.BlockSpec((BBJ, D), lambda i: (i, 0)),
    ],
    out_specs=pl.BlockSpec((BBJ, L, D), lambda i: (i, 0, 0)),
    out_shape=jax.ShapeDtypeStruct((B, L, D), jnp.float32),
    compiler_params=pltpu.CompilerParams(
        dimension_semantics=("arbitrary",)),
)


def kernel(user, tim, traj, mat, traj_len, emb_t, emb_l, emb_u,
           emb_su, emb_sl, emb_tu, emb_tl):
    traj3d = traj.astype(jnp.int32).reshape(NW, 1, ROWS_W)
    tim3d = tim.astype(jnp.int32).reshape(NW, 1, ROWS_W)
    user_i = user.astype(jnp.int32)

    mat_p = jnp.transpose(mat, (1, 2, 3, 0))
    tl4 = traj_len.astype(jnp.int32).reshape(1, 1, 1, B)
    esl_p = emb_sl.T.reshape(1, 1, D, 2)
    esu_p = emb_su.T.reshape(1, 1, D, 2)
    etl_p = emb_tl.T.reshape(1, 1, D, 2)
    etu_p = emb_tu.T.reshape(1, 1, D, 2)
    delta_p = _tc_delta(tl4, mat_p, esl_p, esu_p, etl_p, etu_p)
    delta = jnp.transpose(delta_p, (3, 0, 1, 2))

    embt_pad = jnp.pad(emb_t, ((0, 7), (0, 0)))

    loc_k, tu_k = _sc_kernels()
    rows_t, rows_u = tu_k(tim3d, user_i, embt_pad, emb_u)
    rows_l = loc_k(traj3d, emb_l)

    joint = _tc_joint(
        rows_l.reshape(B, L, D), rows_t.reshape(B, L, D), rows_u)

    return joint, delta


# speedup vs baseline: 1.5279x; 1.1993x over previous
import functools

import jax
import jax.numpy as jnp
from jax import lax
from jax.experimental import pallas as pl
from jax.experimental.pallas import tpu as pltpu
from jax.experimental.pallas import tpu_sc as plsc

HOURS = 24 * 7
SU, SL, TU, TL = 100.0, 0.0, 1000.0, 0.0
B, L, D = 1024, 20, 64
LL = L * L
G = 8

NC, NS = 2, 16
NW = NC * NS
ROWS_W = (B * L) // NW
USR_W = B // NW
K = 32


def _gather_rows(tbl, idx_ref, n, remap, gring, rows, out, out_base,
                 semg, semo):
    ngrp = n // K

    def _load_idx(gbase):
        out = []
        for q in range(K // 16):
            vs = idx_ref[pl.ds(gbase + q * 16, 16)]
            if remap:
                vs = jnp.where(vs == 0, HOURS, vs)
            out.append(vs)
        return out

    def _issue_group(slot_base, gbase):
        grp_v = [(v >> 3) * G for v in _load_idx(gbase)]
        for b in range(K):
            grp = pl.multiple_of(grp_v[b // 16][b % 16], G)
            pltpu.async_copy(tbl.at[pl.ds(grp, G)],
                             gring.at[slot_base + b], semg)

    def _extract_group(slot_base, gbase, half):
        s_v = [v & 7 for v in _load_idx(gbase)]
        for b in range(K):
            pltpu.make_async_copy(
                tbl.at[pl.ds(0, G)], gring.at[slot_base + b], semg).wait()
            s = s_v[b // 16][b % 16]
            for c in range(D // 16):
                sl = pl.ds(c * 16, 16)
                rows[half * K + b, sl] = gring[slot_base + b, s, sl]

    def _out_drain():
        pltpu.make_async_copy(rows.at[pl.ds(0, K)],
                              out.at[pl.ds(out_base, K)], semo).wait()

    _issue_group(0, 0)

    def body(t, carry):
        half = t & 1

        @pl.when(t + 1 < ngrp)
        def _():
            _issue_group((1 - half) * K, (t + 1) * K)

        @pl.when(t >= 2)
        def _():
            _out_drain()

        _extract_group(half * K, t * K, half)
        pltpu.async_copy(rows.at[pl.ds(half * K, K)],
                         out.at[pl.ds(out_base + t * K, K)], semo)
        return carry

    lax.fori_loop(0, ngrp, body, 0, unroll=False)
    for _ in range(min(2, ngrp)):
        _out_drain()


def _extract_table_rows(tblv, idx_ref, n, remap, rows, out, out_base, semo):
    ngrp = n // K

    def _out_drain():
        pltpu.make_async_copy(rows.at[pl.ds(0, K)],
                              out.at[pl.ds(out_base, K)], semo).wait()

    def body(t, carry):
        half = t & 1

        @pl.when(t >= 2)
        def _():
            _out_drain()

        for q in range(K // 16):
            vs = idx_ref[pl.ds(t * K + q * 16, 16)]
            if remap:
                vs = jnp.where(vs == 0, HOURS, vs)
            for b in range(16):
                s = vs[b]
                for c in range(D // 16):
                    sl = pl.ds(c * 16, 16)
                    rows[half * K + q * 16 + b, sl] = tblv[s, sl]
        pltpu.async_copy(rows.at[pl.ds(half * K, K)],
                         out.at[pl.ds(out_base + t * K, K)], semo)
        return carry

    lax.fori_loop(0, ngrp, body, 0, unroll=False)
    for _ in range(min(2, ngrp)):
        _out_drain()


def _sc_loc_body(traj_hbm, embl_hbm, out_l, idx_v, gring, rows, semg, semo):
    wid = lax.axis_index("s") * NC + lax.axis_index("c")
    pltpu.sync_copy(traj_hbm.at[wid], idx_v)
    _gather_rows(embl_hbm, idx_v.at[0], ROWS_W, False, gring, rows,
                 out_l, wid * ROWS_W, semg, semo)


def _sc_tu_body(tim_hbm, user_hbm, embt_hbm, embu_hbm, out_t, out_u,
                idx_v, tblv, gring, rows, semg, semo):
    wid = lax.axis_index("s") * NC + lax.axis_index("c")
    ubase = wid * USR_W

    pltpu.sync_copy(embt_hbm, tblv)
    pltpu.sync_copy(tim_hbm.at[wid], idx_v)
    _extract_table_rows(tblv, idx_v.at[0], ROWS_W, True, rows,
                        out_t, wid * ROWS_W, semo)

    pltpu.sync_copy(user_hbm.at[pl.ds(ubase, USR_W)],
                    idx_v.at[0, pl.ds(0, USR_W)])
    _gather_rows(embu_hbm, idx_v.at[0], USR_W, False, gring, rows,
                 out_u, ubase, semg, semo)


@functools.cache
def _sc_kernels():
    mesh = plsc.VectorSubcoreMesh(
        core_axis_name="c", subcore_axis_name="s",
        num_cores=NC, num_subcores=NS)
    loc = pl.kernel(
        _sc_loc_body,
        mesh=mesh,
        out_type=jax.ShapeDtypeStruct((B * L, D), jnp.float32),
        scratch_types=[
            pltpu.VMEM((1, ROWS_W), jnp.int32),
            pltpu.VMEM((2 * K, G, D), jnp.float32),
            pltpu.VMEM((2 * K, D), jnp.float32),
            pltpu.SemaphoreType.DMA,
            pltpu.SemaphoreType.DMA,
        ],
        compiler_params=pltpu.CompilerParams(use_tc_tiling_on_sc=True),
    )
    tu = pl.kernel(
        _sc_tu_body,
        mesh=mesh,
        out_type=(
            jax.ShapeDtypeStruct((B * L, D), jnp.float32),
            jax.ShapeDtypeStruct((B, D), jnp.float32),
        ),
        scratch_types=[
            pltpu.VMEM((1, ROWS_W), jnp.int32),
            pltpu.VMEM((176, D), jnp.float32),
            pltpu.VMEM((2 * K, G, D), jnp.float32),
            pltpu.VMEM((2 * K, D), jnp.float32),
            pltpu.SemaphoreType.DMA,
            pltpu.SemaphoreType.DMA,
        ],
        compiler_params=pltpu.CompilerParams(use_tc_tiling_on_sc=True),
    )
    return loc, tu


I_BLK = 2


def _delta_body(tl_ref, mat_ref, esl_ref, esu_ref, etl_ref, etu_ref, out_ref):
    i0 = pl.program_id(0) * I_BLK
    tl = tl_ref[...]
    ii = i0 + lax.broadcasted_iota(jnp.int32, (I_BLK, L, 1, 1), 0)
    jj = lax.broadcasted_iota(jnp.int32, (I_BLK, L, 1, 1), 1)
    m = (tl > ii) & (tl > jj)

    esl = esl_ref[...]
    esu = esu_ref[...]
    etl = etl_ref[...]
    etu = etu_ref[...]
    inv_s = 1.0 / (SU - SL)
    inv_t = 1.0 / (TU - TL)
    a = (esu - esl) * inv_s
    b = (etu - etl) * inv_t
    c = (esl * SU - esu * SL) * inv_s + (etl * TU - etu * TL) * inv_t

    wa = jnp.where(m, a[:, :, :, 1:2], a[:, :, :, 0:1])
    wb = jnp.where(m, b[:, :, :, 1:2], b[:, :, :, 0:1])
    wc = jnp.where(m, c[:, :, :, 1:2], c[:, :, :, 0:1])

    ds = mat_ref[:, :, 0:1, :]
    dt = mat_ref[:, :, 1:2, :]
    out_ref[...] = wa * ds + wb * dt + wc


_full4 = lambda shape: pl.BlockSpec(shape, lambda i: (0, 0, 0, 0))

_tc_delta = pl.pallas_call(
    _delta_body,
    grid=(L // I_BLK,),
    in_specs=[
        _full4((1, 1, 1, B)),
        pl.BlockSpec((I_BLK, L, 2, B), lambda i: (i, 0, 0, 0)),
        _full4((1, 1, D, 2)), _full4((1, 1, D, 2)),
        _full4((1, 1, D, 2)), _full4((1, 1, D, 2)),
    ],
    out_specs=pl.BlockSpec((I_BLK, L, D, B), lambda i: (i, 0, 0, 0)),
    out_shape=jax.ShapeDtypeStruct((L, L, D, B), jnp.float32),
    compiler_params=pltpu.CompilerParams(
        dimension_semantics=("arbitrary",)),
)


BBJ = 32


def _joint_body(rl_ref, rt_ref, ru_ref, joint_ref):
    joint_ref[...] = rl_ref[...] + rt_ref[...] + ru_ref[...][:, None, :]


_tc_joint = pl.pallas_call(
    _joint_body,
    grid=(B // BBJ,),
    in_specs=[
        pl.BlockSpec((BBJ, L, D), lambda i: (i, 0, 0)),
        pl.BlockSpec((BBJ, L, D), lambda i: (i, 0, 0)),
        pl.BlockSpec((BBJ, D), lambda i: (i, 0)),
    ],
    out_specs=pl.BlockSpec((BBJ, L, D), lambda i: (i, 0, 0)),
    out_shape=jax.ShapeDtypeStruct((B, L, D), jnp.float32),
    compiler_params=pltpu.CompilerParams(
        dimension_semantics=("arbitrary",)),
)


def kernel(user, tim, traj, mat, traj_len, emb_t, emb_l, emb_u,
           emb_su, emb_sl, emb_tu, emb_tl):
    traj3d = traj.astype(jnp.int32).reshape(NW, 1, ROWS_W)
    tim3d = tim.astype(jnp.int32).reshape(NW, 1, ROWS_W)
    user_i = user.astype(jnp.int32)

    mat_p = jnp.transpose(mat, (1, 2, 3, 0))
    tl4 = traj_len.astype(jnp.int32).reshape(1, 1, 1, B)
    esl_p = emb_sl.T.reshape(1, 1, D, 2)
    esu_p = emb_su.T.reshape(1, 1, D, 2)
    etl_p = emb_tl.T.reshape(1, 1, D, 2)
    etu_p = emb_tu.T.reshape(1, 1, D, 2)
    delta_p = _tc_delta(tl4, mat_p, esl_p, esu_p, etl_p, etu_p)
    delta = jnp.transpose(delta_p, (3, 0, 1, 2))

    embt_pad = jnp.pad(emb_t, ((0, 7), (0, 0)))

    loc_k, tu_k = _sc_kernels()
    rows_t, rows_u = tu_k(tim3d, user_i, embt_pad, emb_u)
    rows_l = loc_k(traj3d, emb_l)

    joint = _tc_joint(
        rows_l.reshape(B, L, D), rows_t.reshape(B, L, D), rows_u)

    return joint, delta
